# baseline (device time: 254781 ns/iter reference)
import numpy as np
import jax
import jax.numpy as jnp
from jax import lax
from jax.experimental import pallas as pl
from jax.experimental.pallas import tpu as pltpu

N_DEV = 4
SQ = 1024
SKV = 1024
HQ = 8
DH = 128
NR = 4
RQ = SQ // NR
D_MODEL = 1024
SCALE = 0.08838834764831843

_blocks = np.arange(SQ // 64)
_order = np.concatenate([np.where(_blocks % 4 == r)[0] for r in range(NR)])
PERM = (_order[:, None] * 64 + np.arange(64)[None, :]).reshape(-1)
INV_PERM = np.argsort(PERM)


def _body(x_ref, wq_ref, wo_ref, k_ref, v_ref, out_ref,
          commq_ref, commo_ref, sq_sems, rq_sems, so_sems, ro_sems):
    my = lax.axis_index("i")
    left = lax.rem(my + N_DEV - 1, N_DEV)
    right = lax.rem(my + 1, N_DEV)

    barrier = pltpu.get_barrier_semaphore()
    for nbr in (left, right):
        pl.semaphore_signal(barrier, inc=1, device_id=(nbr,),
                            device_id_type=pl.DeviceIdType.MESH)
    pl.semaphore_wait(barrier, 2)

    for h in range(N_DEV - 1):
        srcq = wq_ref if h == 0 else commq_ref.at[h - 1]
        srco = wo_ref if h == 0 else commo_ref.at[h - 1]
        rq = pltpu.make_async_remote_copy(
            src_ref=srcq, dst_ref=commq_ref.at[h],
            send_sem=sq_sems.at[h], recv_sem=rq_sems.at[h],
            device_id=(right,), device_id_type=pl.DeviceIdType.MESH,
        )
        ro = pltpu.make_async_remote_copy(
            src_ref=srco, dst_ref=commo_ref.at[h],
            send_sem=so_sems.at[h], recv_sem=ro_sems.at[h],
            device_id=(right,), device_id_type=pl.DeviceIdType.MESH,
        )
        rq.start()
        ro.start()
        rq.wait()
        ro.wait()

    out_ref[...] = jnp.zeros((SQ, D_MODEL), jnp.float32)
    xv = x_ref[...]

    def accum_group(wq_g, wo_g, g):
        head_base = g * HQ
        for r in range(NR):
            xr = xv[r * RQ:(r + 1) * RQ]
            qt = lax.dot_general(
                wq_g, xr, (((1,), (1,)), ((), ())),
                preferred_element_type=jnp.float32).astype(jnp.bfloat16)
            kg = k_ref[pl.ds(head_base, HQ), pl.ds(r * RQ, RQ), :]
            vg = v_ref[pl.ds(head_base, HQ), pl.ds(r * RQ, RQ), :]
            st = lax.dot_general(
                kg, qt, (((2,), (1,)), ((0,), (0,))),
                preferred_element_type=jnp.float32) * SCALE
            m = jnp.max(st, axis=1, keepdims=True)
            e = jnp.exp(st - m)
            w = (e / jnp.sum(e, axis=1, keepdims=True)).astype(jnp.bfloat16)
            ct = lax.dot_general(
                vg, w, (((1,), (1,)), ((0,), (0,))),
                preferred_element_type=jnp.float32).astype(jnp.bfloat16)
            ctf = ct.reshape(HQ * DH, RQ)
            o = lax.dot_general(
                ctf, wo_g, (((0,), (0,)), ((), ())),
                preferred_element_type=jnp.float32)
            out_ref[r * RQ:(r + 1) * RQ, :] += o

    accum_group(wq_ref[...], wo_ref[...], my)
    for h in range(N_DEV - 1):
        g = lax.rem(my + 2 * N_DEV - 1 - h, N_DEV)
        accum_group(commq_ref[h], commo_ref[h], g)


def kernel(x, Wq, K_ext, V_ext, Wo):
    xp = x[0][PERM].astype(jnp.bfloat16)
    wq_t = (Wq.astype(jnp.bfloat16)
            .reshape(D_MODEL, HQ, DH).transpose(1, 0, 2))
    wo_b = Wo.astype(jnp.bfloat16)
    kp = K_ext[0][PERM].transpose(1, 0, 2).astype(jnp.bfloat16)
    vp = V_ext[0][PERM].transpose(1, 0, 2).astype(jnp.bfloat16)

    out = pl.pallas_call(
        _body,
        out_shape=jax.ShapeDtypeStruct((SQ, D_MODEL), jnp.float32),
        in_specs=[pl.BlockSpec(memory_space=pltpu.VMEM)] * 5,
        out_specs=pl.BlockSpec(memory_space=pltpu.VMEM),
        scratch_shapes=[
            pltpu.VMEM((N_DEV - 1, HQ, D_MODEL, DH), jnp.bfloat16),
            pltpu.VMEM((N_DEV - 1, D_MODEL, D_MODEL), jnp.bfloat16),
            pltpu.SemaphoreType.DMA((N_DEV - 1,)),
            pltpu.SemaphoreType.DMA((N_DEV - 1,)),
            pltpu.SemaphoreType.DMA((N_DEV - 1,)),
            pltpu.SemaphoreType.DMA((N_DEV - 1,)),
        ],
        compiler_params=pltpu.CompilerParams(collective_id=0),
    )(xp, wq_t, wo_b, kp, vp)

    return out[INV_PERM][None]


# device time: 125306 ns/iter; 2.0333x vs baseline; 2.0333x over previous
import jax
import jax.numpy as jnp
from jax import lax
from jax.experimental import pallas as pl
from jax.experimental.pallas import tpu as pltpu

N_DEV = 4
SQ = 1024
HQ = 8
DH = 128
NR = 4
RQ = SQ // NR
D_MODEL = 1024
SCALE = 0.08838834764831843


def _body(x_ref, wq_ref, wo_ref, k_ref, v_ref, out_ref,
          commq_ref, commo_ref, sA, rA, sB, rB):
    my = lax.axis_index("i")
    left = lax.rem(my + N_DEV - 1, N_DEV)
    right = lax.rem(my + 1, N_DEV)

    barrier = pltpu.get_barrier_semaphore()
    for nbr in (left, right):
        pl.semaphore_signal(barrier, inc=1, device_id=(nbr,),
                            device_id_type=pl.DeviceIdType.MESH)
    pl.semaphore_wait(barrier, 2)

    def copy(src, dst, ss, rs, dev):
        return pltpu.make_async_remote_copy(
            src_ref=src, dst_ref=dst, send_sem=ss, recv_sem=rs,
            device_id=(dev,), device_id_type=pl.DeviceIdType.MESH)

    a_rq = copy(wq_ref, commq_ref.at[0], sA.at[0], rA.at[0], right)
    a_ro = copy(wo_ref, commo_ref.at[0], sA.at[1], rA.at[1], right)
    a_lq = copy(wq_ref, commq_ref.at[1], sA.at[2], rA.at[2], left)
    a_lo = copy(wo_ref, commo_ref.at[1], sA.at[3], rA.at[3], left)
    for r_ in (a_rq, a_ro, a_lq, a_lo):
        r_.start()

    xrs = [x_ref[:, r].reshape(RQ, D_MODEL) for r in range(NR)]

    def accum_group(wq_g, wo_g, g, first):
        hb = g * HQ
        for r in range(NR):
            qt = lax.dot_general(
                wq_g, xrs[r], (((1,), (1,)), ((), ())),
                preferred_element_type=jnp.float32).astype(jnp.bfloat16)
            kg = k_ref[:, r, :, pl.ds(hb, HQ), :].reshape(RQ, HQ, DH)
            vg = v_ref[:, r, :, pl.ds(hb, HQ), :].reshape(RQ, HQ, DH)
            kgt = jnp.transpose(kg, (1, 0, 2))
            vgt = jnp.transpose(vg, (1, 0, 2))
            st = lax.dot_general(
                kgt, qt, (((2,), (1,)), ((0,), (0,))),
                preferred_element_type=jnp.float32) * SCALE
            m = jnp.max(st, axis=1, keepdims=True)
            e = jnp.exp(st - m)
            w = (e / jnp.sum(e, axis=1, keepdims=True)).astype(jnp.bfloat16)
            ct = lax.dot_general(
                vgt, w, (((1,), (1,)), ((0,), (0,))),
                preferred_element_type=jnp.float32).astype(jnp.bfloat16)
            o = lax.dot_general(
                ct.reshape(HQ * DH, RQ), wo_g, (((0,), (0,)), ((), ())),
                preferred_element_type=jnp.float32)
            ob = o.reshape(NR, 64, D_MODEL)
            if first:
                out_ref[:, r] = ob
            else:
                out_ref[:, r] += ob

    accum_group(wq_ref[...], wo_ref[...], my, first=True)

    a_rq.wait_recv()
    a_ro.wait_recv()
    b_rq = copy(commq_ref.at[0, 0:HQ // 2], commq_ref.at[2, 0:HQ // 2],
                sB.at[0], rB.at[0], right)
    b_ro = copy(commo_ref.at[0, 0:D_MODEL // 2], commo_ref.at[2, 0:D_MODEL // 2],
                sB.at[1], rB.at[1], right)
    b_rq.start()
    b_ro.start()
    accum_group(commq_ref[0], commo_ref[0], left, first=False)

    a_lq.wait_recv()
    a_lo.wait_recv()
    b_lq = copy(commq_ref.at[1, HQ // 2:HQ], commq_ref.at[2, HQ // 2:HQ],
                sB.at[2], rB.at[2], left)
    b_lo = copy(commo_ref.at[1, D_MODEL // 2:D_MODEL],
                commo_ref.at[2, D_MODEL // 2:D_MODEL],
                sB.at[3], rB.at[3], left)
    b_lq.start()
    b_lo.start()
    accum_group(commq_ref[1], commo_ref[1], right, first=False)

    for r_ in (b_rq, b_ro, b_lq, b_lo):
        r_.wait_recv()
    accum_group(commq_ref[2], commo_ref[2], lax.rem(my + 2, N_DEV),
                first=False)

    for r_ in (a_rq, a_ro, a_lq, a_lo, b_rq, b_ro, b_lq, b_lo):
        r_.wait_send()


def kernel(x, Wq, K_ext, V_ext, Wo):
    x_b = x[0].reshape(NR, NR, 64, D_MODEL).astype(jnp.bfloat16)
    wq_t = (Wq.astype(jnp.bfloat16)
            .reshape(D_MODEL, HQ, DH).transpose(1, 0, 2))
    wo_b = Wo.astype(jnp.bfloat16)
    k_b = K_ext[0].reshape(NR, NR, 64, 4 * HQ, DH).astype(jnp.bfloat16)
    v_b = V_ext[0].reshape(NR, NR, 64, 4 * HQ, DH).astype(jnp.bfloat16)

    out = pl.pallas_call(
        _body,
        out_shape=jax.ShapeDtypeStruct((NR, NR, 64, D_MODEL), jnp.float32),
        in_specs=[pl.BlockSpec(memory_space=pltpu.VMEM)] * 5,
        out_specs=pl.BlockSpec(memory_space=pltpu.VMEM),
        scratch_shapes=[
            pltpu.VMEM((3, HQ, D_MODEL, DH), jnp.bfloat16),
            pltpu.VMEM((3, D_MODEL, D_MODEL), jnp.bfloat16),
            pltpu.SemaphoreType.DMA((4,)),
            pltpu.SemaphoreType.DMA((4,)),
            pltpu.SemaphoreType.DMA((4,)),
            pltpu.SemaphoreType.DMA((4,)),
        ],
        compiler_params=pltpu.CompilerParams(collective_id=0),
    )(x_b, wq_t, wo_b, k_b, v_b)

    return out.reshape(1, SQ, D_MODEL)
